# Initial kernel scaffold; baseline (speedup 1.0000x reference)
import jax
import jax.numpy as jnp
from jax.experimental import pallas as pl


def kernel(hidden_states, W_router, b_router, W_bvh1, b_bvh1, W_bvh2, b_bvh2, per_expert_scale):
    raise NotImplementedError("write your pallas kernel here")


# speedup vs baseline: 1.1140x; 1.1140x over previous
import functools

import jax
import jax.numpy as jnp
from jax.experimental import pallas as pl

T = 4096
D = 4096
E = 64
H = 1024
TOP_K = 8
N_CAND = 16

BT = 256

_PREC = jax.lax.Precision.DEFAULT


def _peel_max(v, alive, iota):
    neg = jnp.float32(-jnp.inf)
    vm = jnp.where(alive, v, neg)
    m = jnp.max(vm, axis=-1, keepdims=True)
    eq = (vm == m)
    idx = jnp.min(jnp.where(eq, iota, E), axis=-1, keepdims=True)
    taken = iota == idx
    return m, idx, alive & ~taken


def _router_block(x_ref, wr_ref, br_ref, w1_ref, b1_ref, w2_ref, b2_ref,
                  pes_ref, probs_ref, w_ref, i_ref):
    x = x_ref[...]

    logits = jnp.dot(x, wr_ref[...], preferred_element_type=jnp.float32,
                     precision=_PREC) + br_ref[...]
    m = jnp.max(logits, axis=-1, keepdims=True)
    e = jnp.exp(logits - m)
    probs = e / jnp.sum(e, axis=-1, keepdims=True)
    probs_ref[...] = probs

    h1 = jnp.dot(x, w1_ref[...], preferred_element_type=jnp.float32,
                 precision=_PREC) + b1_ref[...]
    h1 = jnp.maximum(h1, 0.0)
    bvh = jnp.dot(h1, w2_ref[...], preferred_element_type=jnp.float32,
                  precision=_PREC) + b2_ref[...]

    iota = jax.lax.broadcasted_iota(jnp.int32, (BT, E), 1)

    alive = jnp.ones((BT, E), dtype=jnp.bool_)
    for _ in range(N_CAND):
        _, _, alive = _peel_max(bvh, alive, iota)
    cand = ~alive

    mp = jnp.where(cand, probs, -1.0)
    alive = jnp.ones((BT, E), dtype=jnp.bool_)
    vals, idxs = [], []
    for _ in range(TOP_K):
        v, ix, alive = _peel_max(mp, alive, iota)
        vals.append(v)
        idxs.append(ix)
    vals = jnp.concatenate(vals, axis=1)
    idx = jnp.concatenate(idxs, axis=1)

    onehot = (idx[:, :, None] ==
              jax.lax.broadcasted_iota(jnp.int32, (1, 1, E), 2))
    scale = jnp.sum(jnp.where(onehot, pes_ref[...][:, None, :], 0.0), axis=2)

    w = vals / jnp.sum(vals, axis=-1, keepdims=True)
    w_ref[...] = w * scale
    i_ref[...] = idx


@jax.jit
def kernel(hidden_states, W_router, b_router, W_bvh1, b_bvh1, W_bvh2, b_bvh2,
           per_expert_scale):
    x = hidden_states.reshape(T, D)
    grid = (T // BT,)
    full = lambda i: (0, 0)
    out = pl.pallas_call(
        _router_block,
        grid=grid,
        in_specs=[
            pl.BlockSpec((BT, D), lambda i: (i, 0)),
            pl.BlockSpec((D, E), full),
            pl.BlockSpec((1, E), full),
            pl.BlockSpec((D, H), full),
            pl.BlockSpec((1, H), full),
            pl.BlockSpec((H, E), full),
            pl.BlockSpec((1, E), full),
            pl.BlockSpec((1, E), full),
        ],
        out_specs=[
            pl.BlockSpec((BT, E), lambda i: (i, 0)),
            pl.BlockSpec((BT, TOP_K), lambda i: (i, 0)),
            pl.BlockSpec((BT, TOP_K), lambda i: (i, 0)),
        ],
        out_shape=[
            jax.ShapeDtypeStruct((T, E), jnp.float32),
            jax.ShapeDtypeStruct((T, TOP_K), jnp.float32),
            jax.ShapeDtypeStruct((T, TOP_K), jnp.int32),
        ],
    )(x, W_router, b_router.reshape(1, E), W_bvh1, b_bvh1.reshape(1, H),
      W_bvh2, b_bvh2.reshape(1, E), per_expert_scale.reshape(1, E))
    return (out[0], out[1], out[2])


# speedup vs baseline: 1.4607x; 1.3112x over previous
import functools

import jax
import jax.numpy as jnp
from jax.experimental import pallas as pl

T = 4096
D = 4096
E = 64
H = 1024
TOP_K = 8
N_CAND = 16

BT = 256

_PREC = jax.lax.Precision.DEFAULT


def _peel_max(v, alive, iota):
    neg = jnp.float32(-jnp.inf)
    vm = jnp.where(alive, v, neg)
    m = jnp.max(vm, axis=-1, keepdims=True)
    eq = (vm == m)
    idx = jnp.min(jnp.where(eq, iota, E), axis=-1, keepdims=True)
    taken = iota == idx
    return m, idx, alive & ~taken


def _router_block(x_ref, wr_ref, br_ref, w1_ref, b1_ref, w2_ref, b2_ref,
                  pes_ref, probs_ref, w_ref, i_ref):
    x = x_ref[...]

    logits = jnp.dot(x, wr_ref[...], preferred_element_type=jnp.float32,
                     precision=_PREC) + br_ref[...]
    m = jnp.max(logits, axis=-1, keepdims=True)
    e = jnp.exp(logits - m)
    probs = e / jnp.sum(e, axis=-1, keepdims=True)
    probs_ref[...] = probs

    h1 = jnp.dot(x, w1_ref[...], preferred_element_type=jnp.float32,
                 precision=_PREC) + b1_ref[...]
    h1 = jnp.maximum(h1, 0.0)
    bvh = jnp.dot(h1, w2_ref[...], preferred_element_type=jnp.float32,
                  precision=_PREC) + b2_ref[...]

    iota = jax.lax.broadcasted_iota(jnp.int32, (BT, E), 1)

    neg = jnp.float32(-jnp.inf)
    vm = bvh
    for _ in range(N_CAND):
        m = jnp.max(vm, axis=-1, keepdims=True)
        vm = jnp.where(vm == m, neg, vm)
    cand = vm == neg

    mp = jnp.where(cand, probs, -1.0)
    alive = jnp.ones((BT, E), dtype=jnp.bool_)
    vals, idxs = [], []
    for _ in range(TOP_K):
        v, ix, alive = _peel_max(mp, alive, iota)
        vals.append(v)
        idxs.append(ix)
    vals = jnp.concatenate(vals, axis=1)
    idx = jnp.concatenate(idxs, axis=1)

    onehot = (idx[:, :, None] ==
              jax.lax.broadcasted_iota(jnp.int32, (1, 1, E), 2))
    scale = jnp.sum(jnp.where(onehot, pes_ref[...][:, None, :], 0.0), axis=2)

    w = vals / jnp.sum(vals, axis=-1, keepdims=True)
    w_ref[...] = w * scale
    i_ref[...] = idx


@jax.jit
def kernel(hidden_states, W_router, b_router, W_bvh1, b_bvh1, W_bvh2, b_bvh2,
           per_expert_scale):
    x = hidden_states.reshape(T, D)
    grid = (T // BT,)
    full = lambda i: (0, 0)
    out = pl.pallas_call(
        _router_block,
        grid=grid,
        in_specs=[
            pl.BlockSpec((BT, D), lambda i: (i, 0)),
            pl.BlockSpec((D, E), full),
            pl.BlockSpec((1, E), full),
            pl.BlockSpec((D, H), full),
            pl.BlockSpec((1, H), full),
            pl.BlockSpec((H, E), full),
            pl.BlockSpec((1, E), full),
            pl.BlockSpec((1, E), full),
        ],
        out_specs=[
            pl.BlockSpec((BT, E), lambda i: (i, 0)),
            pl.BlockSpec((BT, TOP_K), lambda i: (i, 0)),
            pl.BlockSpec((BT, TOP_K), lambda i: (i, 0)),
        ],
        out_shape=[
            jax.ShapeDtypeStruct((T, E), jnp.float32),
            jax.ShapeDtypeStruct((T, TOP_K), jnp.float32),
            jax.ShapeDtypeStruct((T, TOP_K), jnp.int32),
        ],
    )(x, W_router, b_router.reshape(1, E), W_bvh1, b_bvh1.reshape(1, H),
      W_bvh2, b_bvh2.reshape(1, E), per_expert_scale.reshape(1, E))
    return (out[0], out[1], out[2])


# speedup vs baseline: 2.2825x; 1.5626x over previous
import functools

import jax
import jax.numpy as jnp
from jax.experimental import pallas as pl

T = 4096
D = 4096
E = 64
H = 1024
TOP_K = 8
N_CAND = 16

BT = 256

_PREC = jax.lax.Precision.DEFAULT


def _router_block(x_ref, wr_ref, br_ref, w1_ref, b1_ref, w2_ref, b2_ref,
                  pes_ref, probs_ref, w_ref, i_ref):
    x = x_ref[...]

    logits = jnp.dot(x, wr_ref[...], preferred_element_type=jnp.float32,
                     precision=_PREC) + br_ref[...]
    lt = logits.T
    m = jnp.max(lt, axis=0, keepdims=True)
    e = jnp.exp(lt - m)
    pt = e / jnp.sum(e, axis=0, keepdims=True)
    probs_ref[...] = pt.T

    h1 = jnp.dot(x, w1_ref[...], preferred_element_type=jnp.float32,
                 precision=_PREC) + b1_ref[...]
    h1 = jnp.maximum(h1, 0.0)
    bvh = jnp.dot(h1, w2_ref[...], preferred_element_type=jnp.float32,
                  precision=_PREC) + b2_ref[...]
    bt = bvh.T

    neg = jnp.float32(-jnp.inf)
    vm = bt
    for _ in range(N_CAND):
        mm = jnp.max(vm, axis=0, keepdims=True)
        vm = jnp.where(vm == mm, neg, vm)
    cand = vm == neg

    iota_e = jax.lax.broadcasted_iota(jnp.int32, (E, BT), 0)
    pes_col = pes_ref[...]
    mp = jnp.where(cand, pt, -1.0)
    vals, idxs, scales = [], [], []
    for _ in range(TOP_K):
        mm = jnp.max(mp, axis=0, keepdims=True)
        eq = mp == mm
        ix = jnp.min(jnp.where(eq, iota_e, E), axis=0, keepdims=True)
        taken = iota_e == ix
        mp = jnp.where(taken, neg, mp)
        vals.append(mm)
        idxs.append(ix)
        scales.append(jnp.sum(jnp.where(taken, pes_col, 0.0), axis=0,
                              keepdims=True))
    vals = jnp.concatenate(vals, axis=0)
    idx = jnp.concatenate(idxs, axis=0)
    scale = jnp.concatenate(scales, axis=0)

    w = vals / jnp.sum(vals, axis=0, keepdims=True)
    w_ref[...] = (w * scale).T
    i_ref[...] = idx.T


@jax.jit
def kernel(hidden_states, W_router, b_router, W_bvh1, b_bvh1, W_bvh2, b_bvh2,
           per_expert_scale):
    x = hidden_states.reshape(T, D)
    grid = (T // BT,)
    full = lambda i: (0, 0)
    out = pl.pallas_call(
        _router_block,
        grid=grid,
        in_specs=[
            pl.BlockSpec((BT, D), lambda i: (i, 0)),
            pl.BlockSpec((D, E), full),
            pl.BlockSpec((1, E), full),
            pl.BlockSpec((D, H), full),
            pl.BlockSpec((1, H), full),
            pl.BlockSpec((H, E), full),
            pl.BlockSpec((1, E), full),
            pl.BlockSpec((E, 1), full),
        ],
        out_specs=[
            pl.BlockSpec((BT, E), lambda i: (i, 0)),
            pl.BlockSpec((BT, TOP_K), lambda i: (i, 0)),
            pl.BlockSpec((BT, TOP_K), lambda i: (i, 0)),
        ],
        out_shape=[
            jax.ShapeDtypeStruct((T, E), jnp.float32),
            jax.ShapeDtypeStruct((T, TOP_K), jnp.float32),
            jax.ShapeDtypeStruct((T, TOP_K), jnp.int32),
        ],
    )(x, W_router, b_router.reshape(1, E), W_bvh1, b_bvh1.reshape(1, H),
      W_bvh2, b_bvh2.reshape(1, E), per_expert_scale.reshape(E, 1))
    return (out[0], out[1], out[2])


# speedup vs baseline: 2.4302x; 1.0647x over previous
import functools

import jax
import jax.numpy as jnp
from jax.experimental import pallas as pl

T = 4096
D = 4096
E = 64
H = 1024
TOP_K = 8
N_CAND = 16

BT = 512

_PREC = jax.lax.Precision.DEFAULT


def _router_block(x_ref, wr_ref, br_ref, w1_ref, b1_ref, w2_ref, b2_ref,
                  pes_ref, probs_ref, w_ref, i_ref):
    x = x_ref[...]

    logits = jnp.dot(x, wr_ref[...], preferred_element_type=jnp.float32,
                     precision=_PREC) + br_ref[...]
    lt = logits.T
    m = jnp.max(lt, axis=0, keepdims=True)
    e = jnp.exp(lt - m)
    pt = e / jnp.sum(e, axis=0, keepdims=True)
    probs_ref[...] = pt.T

    h1 = jnp.dot(x, w1_ref[...], preferred_element_type=jnp.float32,
                 precision=_PREC) + b1_ref[...]
    h1 = jnp.maximum(h1, 0.0)
    bvh = jnp.dot(h1, w2_ref[...], preferred_element_type=jnp.float32,
                  precision=_PREC) + b2_ref[...]
    bt = bvh.T

    neg = jnp.float32(-jnp.inf)
    vm = bt
    for _ in range(N_CAND):
        mm = jnp.max(vm, axis=0, keepdims=True)
        vm = jnp.where(vm == mm, neg, vm)
    cand = vm == neg

    iota_e = jax.lax.broadcasted_iota(jnp.int32, (E, BT), 0)
    pes_col = pes_ref[...]
    mp = jnp.where(cand, pt, -1.0)
    vals, idxs, scales = [], [], []
    for _ in range(TOP_K):
        mm = jnp.max(mp, axis=0, keepdims=True)
        eq = mp == mm
        ix = jnp.min(jnp.where(eq, iota_e, E), axis=0, keepdims=True)
        taken = iota_e == ix
        mp = jnp.where(taken, neg, mp)
        vals.append(mm)
        idxs.append(ix)
        scales.append(jnp.sum(jnp.where(taken, pes_col, 0.0), axis=0,
                              keepdims=True))
    vals = jnp.concatenate(vals, axis=0)
    idx = jnp.concatenate(idxs, axis=0)
    scale = jnp.concatenate(scales, axis=0)

    w = vals / jnp.sum(vals, axis=0, keepdims=True)
    w_ref[...] = (w * scale).T
    i_ref[...] = idx.T


@jax.jit
def kernel(hidden_states, W_router, b_router, W_bvh1, b_bvh1, W_bvh2, b_bvh2,
           per_expert_scale):
    x = hidden_states.reshape(T, D)
    grid = (T // BT,)
    full = lambda i: (0, 0)
    out = pl.pallas_call(
        _router_block,
        grid=grid,
        in_specs=[
            pl.BlockSpec((BT, D), lambda i: (i, 0)),
            pl.BlockSpec((D, E), full),
            pl.BlockSpec((1, E), full),
            pl.BlockSpec((D, H), full),
            pl.BlockSpec((1, H), full),
            pl.BlockSpec((H, E), full),
            pl.BlockSpec((1, E), full),
            pl.BlockSpec((E, 1), full),
        ],
        out_specs=[
            pl.BlockSpec((BT, E), lambda i: (i, 0)),
            pl.BlockSpec((BT, TOP_K), lambda i: (i, 0)),
            pl.BlockSpec((BT, TOP_K), lambda i: (i, 0)),
        ],
        out_shape=[
            jax.ShapeDtypeStruct((T, E), jnp.float32),
            jax.ShapeDtypeStruct((T, TOP_K), jnp.float32),
            jax.ShapeDtypeStruct((T, TOP_K), jnp.int32),
        ],
    )(x, W_router, b_router.reshape(1, E), W_bvh1, b_bvh1.reshape(1, H),
      W_bvh2, b_bvh2.reshape(1, E), per_expert_scale.reshape(E, 1))
    return (out[0], out[1], out[2])
